# Initial kernel scaffold; baseline (speedup 1.0000x reference)
import jax
import jax.numpy as jnp
from jax.experimental import pallas as pl


def kernel(input_ids, embed_table):
    raise NotImplementedError("write your pallas kernel here")


# speedup vs baseline: 3.0638x; 3.0638x over previous
import functools

import jax
import jax.numpy as jnp
from jax import lax
from jax.experimental import pallas as pl
from jax.experimental.pallas import tpu as pltpu
from jax.experimental.pallas import tpu_sc as plsc

_D = 64
_CHUNK = 512
_NW = 32


def _sc_embedding_gather(idx_flat, table, b_total):
    b_per_w = b_total // _NW
    chunks = b_per_w // _CHUNK
    assert chunks % 2 == 0 and chunks >= 6
    mesh = plsc.VectorSubcoreMesh(core_axis_name="c", subcore_axis_name="s")

    @functools.partial(
        pl.kernel,
        out_type=jax.ShapeDtypeStruct((b_total, _D), jnp.float32),
        mesh=mesh,
        compiler_params=pltpu.CompilerParams(use_tc_tiling_on_sc=False),
        scratch_types=[
            pltpu.VMEM((_CHUNK,), jnp.int32),
            pltpu.VMEM((_CHUNK,), jnp.int32),
            pltpu.VMEM((_CHUNK, _D), jnp.float32),
            pltpu.VMEM((_CHUNK, _D), jnp.float32),
            pltpu.SemaphoreType.DMA,
            pltpu.SemaphoreType.DMA,
            pltpu.SemaphoreType.DMA,
            pltpu.SemaphoreType.DMA,
            pltpu.SemaphoreType.DMA,
            pltpu.SemaphoreType.DMA,
        ],
    )
    def k(idx_hbm, table_hbm, out_hbm, idx0, idx1, rows0, rows1,
          si0, si1, sg0, sg1, so0, so1):
        wid = lax.axis_index("s") * 2 + lax.axis_index("c")
        base = wid * b_per_w
        idx_v = (idx0, idx1)
        rows_v = (rows0, rows1)
        sem_i = (si0, si1)
        sem_g = (sg0, sg1)
        sem_o = (so0, so1)

        def idx_slice(i):
            return idx_hbm.at[pl.ds(base + i * _CHUNK, _CHUNK)]

        def out_slice(i):
            return out_hbm.at[pl.ds(base + i * _CHUNK, _CHUNK)]

        def emit(i, b, do_out_wait=True, do_idx=True, do_gather=True):
            b1 = 1 - b
            pltpu.make_async_copy(table_hbm.at[idx_v[b]], rows_v[b],
                                  sem_g[b]).wait()
            pltpu.async_copy(rows_v[b], out_slice(i), sem_o[b])
            if do_idx:
                pltpu.async_copy(idx_slice(i + 2), idx_v[b], sem_i[b])
            if do_out_wait:
                pltpu.make_async_copy(rows_v[b1], out_slice(i), sem_o[b1]).wait()
            if do_gather:
                pltpu.make_async_copy(idx_slice(i), idx_v[b1], sem_i[b1]).wait()
                pltpu.async_copy(table_hbm.at[idx_v[b1]], rows_v[b1], sem_g[b1])

        pltpu.async_copy(idx_slice(0), idx_v[0], sem_i[0])
        pltpu.async_copy(idx_slice(1), idx_v[1], sem_i[1])
        pltpu.make_async_copy(idx_slice(0), idx_v[0], sem_i[0]).wait()
        pltpu.async_copy(table_hbm.at[idx_v[0]], rows_v[0], sem_g[0])

        emit(0, 0, do_out_wait=False)
        emit(1, 1)

        def body(g, carry):
            i = 2 + 2 * g
            emit(i, 0)
            emit(i + 1, 1)
            return carry

        lax.fori_loop(0, (chunks - 4) // 2, body, 0)

        emit(chunks - 2, 0, do_idx=False)
        emit(chunks - 1, 1, do_idx=False, do_gather=False)
        pltpu.make_async_copy(rows_v[1], out_slice(chunks - 1), sem_o[1]).wait()

    return k(idx_flat, table)


def kernel(input_ids, embed_table):
    b, t = input_ids.shape
    idx_flat = input_ids.reshape(b * t).astype(jnp.int32)
    out = _sc_embedding_gather(idx_flat, embed_table, b * t)
    return out.reshape(b, t, _D)
